# baseline (speedup 1.0000x reference)
import jax
import jax.numpy as jnp
from jax.experimental import pallas as pl


def _gcn_block_kernel(adj_ref, x_ref, w_ref, out_ref):
    t = jnp.dot(
        adj_ref[...].astype(jnp.bfloat16),
        x_ref[...],
        preferred_element_type=jnp.float32,
    )
    out_ref[...] = jnp.dot(t, w_ref[...], preferred_element_type=jnp.float32)


@jax.jit
def kernel(x, adj, weight):
    n, d_in = x.shape
    d_out = weight.shape[1]
    bm = 200

    return pl.pallas_call(
        _gcn_block_kernel,
        grid=(n // bm,),
        in_specs=[
            pl.BlockSpec((bm, n), lambda i: (i, 0)),
            pl.BlockSpec((n, d_in), lambda i: (0, 0)),
            pl.BlockSpec((d_in, d_out), lambda i: (0, 0)),
        ],
        out_specs=pl.BlockSpec((bm, d_out), lambda i: (i, 0)),
        out_shape=jax.ShapeDtypeStruct((n, d_out), jnp.float32),
    )(adj, x.astype(jnp.bfloat16), weight)


# speedup vs baseline: 1.0434x; 1.0434x over previous
import jax
import jax.numpy as jnp
from jax.experimental import pallas as pl


def _gcn_block_kernel(adj1_ref, adj2_ref, x_ref, w_ref, out_ref):
    bm = adj1_ref.shape[0]
    t1 = jnp.dot(adj1_ref[...], x_ref[...], preferred_element_type=jnp.float32)
    t2 = jnp.dot(adj2_ref[...], x_ref[...], preferred_element_type=jnp.float32)
    out_ref[:bm, :] = jnp.dot(t1, w_ref[...], preferred_element_type=jnp.float32)
    out_ref[bm:, :] = jnp.dot(t2, w_ref[...], preferred_element_type=jnp.float32)


@jax.jit
def kernel(x, adj, weight):
    n, d_in = x.shape
    d_out = weight.shape[1]
    bm = 200

    return pl.pallas_call(
        _gcn_block_kernel,
        grid=(n // (2 * bm),),
        in_specs=[
            pl.BlockSpec((bm, n), lambda i: (2 * i, 0)),
            pl.BlockSpec((bm, n), lambda i: (2 * i + 1, 0)),
            pl.BlockSpec((n, d_in), lambda i: (0, 0)),
            pl.BlockSpec((d_in, d_out), lambda i: (0, 0)),
        ],
        out_specs=pl.BlockSpec((2 * bm, d_out), lambda i: (i, 0)),
        out_shape=jax.ShapeDtypeStruct((n, d_out), jnp.float32),
    )(adj, adj, x, weight)
